# baseline (speedup 1.0000x reference)
import functools

import jax
import jax.numpy as jnp
from jax import lax
from jax.experimental import pallas as pl
from jax.experimental.pallas import tpu as pltpu
import jax.experimental.pallas.tpu_sc as plsc

N = 10000
E = 320000
D = 128
G = 32
EPS = 1e-5

NC = 2
NS = 16
NW = NC * NS
K = 112
NCHUNK = 90
EP = NCHUNK * K
EPAD = NW * EP
RI = 4
IA = 2
STRIPE = 624
TAIL = N - NS * STRIPE
ZB = 16

_mesh = plsc.VectorSubcoreMesh(
    core_axis_name="c", subcore_axis_name="s", num_cores=NC, num_subcores=NS)


@functools.partial(
    pl.kernel,
    out_type=jax.ShapeDtypeStruct((NC, N, D), jnp.float32),
    mesh=_mesh,
    scratch_types=[
        pltpu.VMEM((RI, 3, K), jnp.int32),
        pltpu.VMEM((2, K, D), jnp.float32),
        pltpu.VMEM_SHARED((N, D), jnp.float32),
        [pltpu.SemaphoreType.DMA] * RI,
        [pltpu.SemaphoreType.DMA] * 2,
        [pltpu.SemaphoreType.DMA] * 2,
    ],
)
def _sc_aggregate(x_hbm, idx_hbm, out_hbm,
                  idx_v, rows_v, acc_sh, isem, gsem, ssem):
    c = lax.axis_index("c")
    s = lax.axis_index("s")
    w = s * NC + c

    zvec = jnp.zeros((16,), jnp.float32)
    zero_v = rows_v.at[0].at[pl.ds(0, ZB)]

    def _zero_row(r, _):
        for j in range(D // 16):
            rows_v[0, r, pl.ds(j * 16, 16)] = zvec
        return _

    lax.fori_loop(0, ZB, _zero_row, 0)

    def _zero_acc(i, _):
        pltpu.sync_copy(zero_v, acc_sh.at[pl.ds(s * STRIPE + i * ZB, ZB)])
        return _

    lax.fori_loop(0, STRIPE // ZB, _zero_acc, 0)

    @pl.when(s == 0)
    def _zero_tail():
        pltpu.sync_copy(zero_v, acc_sh.at[pl.ds(NS * STRIPE, TAIL)])

    plsc.subcore_barrier()

    def _istart(j, m):
        pltpu.async_copy(idx_hbm.at[w, j], idx_v.at[m], isem[m])

    def _iwait(j, m):
        pltpu.make_async_copy(idx_hbm.at[w, j], idx_v.at[m], isem[m]).wait()

    x_my = x_hbm.at[c]

    def _gstart(m, b):
        pltpu.async_copy(x_my.at[idx_v.at[m].at[0]], rows_v.at[b], gsem[b])

    def _gwait(m, b):
        pltpu.make_async_copy(
            x_my.at[idx_v.at[m].at[0]], rows_v.at[b], gsem[b]).wait()

    def _sstart(m, b):
        pltpu.async_copy(rows_v.at[b], acc_sh.at[idx_v.at[m].at[1]], ssem[b],
                         add=True)

    def _swait(m, b):
        pltpu.make_async_copy(
            rows_v.at[b], acc_sh.at[idx_v.at[m].at[1]], ssem[b]).wait()

    def _scale(m, b):
        row_b = rows_v.at[b]

        def _grp(g, _):
            av = lax.bitcast_convert_type(
                idx_v[m, 2, pl.ds(g * 16, 16)], jnp.float32)

            for j in range(16):
                a = av[j]
                e = g * 16 + j
                for q in range(D // 16):
                    row_b[e, pl.ds(q * 16, 16)] = (
                        row_b[e, pl.ds(q * 16, 16)] * a)
            return _

        lax.fori_loop(0, K // 16, _grp, 0)

    def _iter(i, m, b, do_swait, do_istart, do_gstart):
        if do_swait:
            _swait((m + RI - 1) % RI, 1 - b)
        if do_istart:
            _istart(i + IA, (m + IA) % RI)
        if do_gstart:
            _iwait(i + 1, (m + 1) % RI)
            _gstart((m + 1) % RI, 1 - b)
        _gwait(m, b)
        _scale(m, b)
        _sstart(m, b)

    for j in range(IA):
        _istart(jnp.int32(j), j)
    _iwait(jnp.int32(0), 0)
    _gstart(0, 0)

    _iter(jnp.int32(0), 0, 0, False, True, True)

    ngroups = (NCHUNK - 1 - IA) // 4

    def _group(g, _):
        for t in range(4):
            i = 1 + g * 4 + t
            _iter(i, (1 + t) % RI, (1 + t) % 2, True, True, True)
        return _

    lax.fori_loop(0, ngroups, _group, 0)

    hi = 1 + ngroups * 4
    for i in range(hi, NCHUNK):
        _iter(jnp.int32(i), i % RI, i % 2,
              True, i + IA < NCHUNK, i + 1 < NCHUNK)

    _swait((NCHUNK - 1) % RI, (NCHUNK - 1) % 2)

    plsc.subcore_barrier()

    pltpu.sync_copy(acc_sh.at[pl.ds(s * STRIPE, STRIPE)],
                    out_hbm.at[c, pl.ds(s * STRIPE, STRIPE)])

    @pl.when(s == 0)
    def _copy_tail():
        pltpu.sync_copy(acc_sh.at[pl.ds(NS * STRIPE, TAIL)],
                        out_hbm.at[c, pl.ds(NS * STRIPE, TAIL)])


def _tc_body(x_ref, p_ref, bi_col_ref, bi_row_ref, wrel_t_ref, brel_ref,
             wroot_t_ref, gnw_ref, gnb_ref, gnms_ref, out_ref):
    x = x_ref[...]
    agg = p_ref[0] + p_ref[1]
    h = (jnp.dot(agg, wrel_t_ref[...], preferred_element_type=jnp.float32)
         + brel_ref[...]
         + jnp.dot(x, wroot_t_ref[...], preferred_element_type=jnp.float32))
    h = jnp.maximum(h, 0.0)

    bi_col = bi_col_ref[...]
    bi_row = bi_row_ref[...]
    mt = (lax.broadcasted_iota(jnp.int32, (G, N), 0) == bi_row)
    mt = mt.astype(jnp.float32)
    m = (lax.broadcasted_iota(jnp.int32, (N, G), 1) == bi_col)
    m = m.astype(jnp.float32)

    cnt = jnp.maximum(jnp.sum(mt, axis=1, keepdims=True), 1.0)
    mean = jnp.dot(mt, h, preferred_element_type=jnp.float32) / cnt
    ms = mean * gnms_ref[...]
    out = h - jnp.dot(m, ms, preferred_element_type=jnp.float32)
    var = jnp.dot(mt, out * out, preferred_element_type=jnp.float32) / cnt
    rstd = 1.0 / jnp.sqrt(var + EPS)
    out = out * jnp.dot(m, rstd, preferred_element_type=jnp.float32)
    out_ref[...] = out * gnw_ref[...] + gnb_ref[...]


def kernel(x, edge_index, edge_attr, batch_index, W_rel, b_rel, W_root,
           gn_weight, gn_bias, gn_mean_scale):
    pad = EPAD - E
    src = jnp.pad(edge_index[0], (0, pad)).reshape(NW, NCHUNK, 1, K)
    dst = jnp.pad(edge_index[1], (0, pad)).reshape(NW, NCHUNK, 1, K)
    attr = lax.bitcast_convert_type(jnp.pad(edge_attr, (0, pad)), jnp.int32)
    attr = attr.reshape(NW, NCHUNK, 1, K)
    idx = jnp.concatenate([src, dst, attr], axis=2)
    x2 = jnp.stack([x, x])
    partials = _sc_aggregate(x2, idx)

    bi_col = batch_index.reshape(N, 1)
    bi_row = batch_index.reshape(1, N)
    out = pl.pallas_call(
        _tc_body,
        out_shape=jax.ShapeDtypeStruct((N, D), jnp.float32),
    )(x, partials, bi_col, bi_row, W_rel.T, b_rel.reshape(1, D), W_root.T,
      gn_weight.reshape(1, D), gn_bias.reshape(1, D),
      gn_mean_scale.reshape(1, D))
    return out


# speedup vs baseline: 1.0602x; 1.0602x over previous
import functools

import jax
import jax.numpy as jnp
from jax import lax
from jax.experimental import pallas as pl
from jax.experimental.pallas import tpu as pltpu
import jax.experimental.pallas.tpu_sc as plsc

N = 10000
E = 320000
D = 128
G = 32
EPS = 1e-5

NC = 2
NS = 16
NW = NC * NS
K = 112
NCHUNK = 90
EP = NCHUNK * K
EPAD = NW * EP
CB = 3
NB = NCHUNK // CB
STRIPE = 624
TAIL = N - NS * STRIPE
ZB = 16

_mesh = plsc.VectorSubcoreMesh(
    core_axis_name="c", subcore_axis_name="s", num_cores=NC, num_subcores=NS)


@functools.partial(
    pl.kernel,
    out_type=jax.ShapeDtypeStruct((NC, N, D), jnp.float32),
    mesh=_mesh,
    scratch_types=[
        pltpu.VMEM((2, 3, CB, K), jnp.int32),
        pltpu.VMEM((2, K, D), jnp.float32),
        pltpu.VMEM_SHARED((N, D), jnp.float32),
        [pltpu.SemaphoreType.DMA] * 2,
        [pltpu.SemaphoreType.DMA] * 2,
        [pltpu.SemaphoreType.DMA] * 2,
    ],
)
def _sc_aggregate(x_hbm, idx_hbm, out_hbm,
                  idx_v, rows_v, acc_sh, isem, gsem, ssem):
    c = lax.axis_index("c")
    s = lax.axis_index("s")
    w = s * NC + c

    zvec = jnp.zeros((16,), jnp.float32)
    zero_v = rows_v.at[0].at[pl.ds(0, ZB)]

    def _zero_row(r, _):
        for j in range(D // 16):
            rows_v[0, r, pl.ds(j * 16, 16)] = zvec
        return _

    lax.fori_loop(0, ZB, _zero_row, 0)

    def _zero_acc(i, _):
        pltpu.sync_copy(zero_v, acc_sh.at[pl.ds(s * STRIPE + i * ZB, ZB)])
        return _

    lax.fori_loop(0, STRIPE // ZB, _zero_acc, 0)

    @pl.when(s == 0)
    def _zero_tail():
        pltpu.sync_copy(zero_v, acc_sh.at[pl.ds(NS * STRIPE, TAIL)])

    plsc.subcore_barrier()

    def _istart(nb, m):
        pltpu.async_copy(idx_hbm.at[w, nb], idx_v.at[m], isem[m])

    def _iwait(nb, m):
        pltpu.make_async_copy(idx_hbm.at[w, nb], idx_v.at[m], isem[m]).wait()

    def _gstart(m, sub, b):
        pltpu.async_copy(x_hbm.at[idx_v.at[m].at[0, sub]], rows_v.at[b],
                         gsem[b])

    def _gwait(m, sub, b):
        pltpu.make_async_copy(
            x_hbm.at[idx_v.at[m].at[0, sub]], rows_v.at[b], gsem[b]).wait()

    def _sstart(m, sub, b):
        pltpu.async_copy(rows_v.at[b], acc_sh.at[idx_v.at[m].at[1, sub]],
                         ssem[b], add=True)

    def _swait(m, sub, b):
        pltpu.make_async_copy(
            rows_v.at[b], acc_sh.at[idx_v.at[m].at[1, sub]], ssem[b]).wait()

    def _scale(m, sub, b):
        row_b = rows_v.at[b]

        def _grp(g, _):
            av = lax.bitcast_convert_type(
                idx_v[m, 2, sub, pl.ds(g * 16, 16)], jnp.float32)

            for j in range(16):
                a = av[j]
                e = g * 16 + j
                for q in range(D // 16):
                    row_b[e, pl.ds(q * 16, 16)] = (
                        row_b[e, pl.ds(q * 16, 16)] * a)
            return _

        lax.fori_loop(0, K // 16, _grp, 0)

    def _iter(i, pos, do_swait, do_istart, do_gstart, do_iwait):
        sub = pos % CB
        m = (pos // CB) % 2
        b = pos % 2
        if do_swait:
            _swait(((pos + 5) // CB) % 2, (pos + 5) % CB, 1 - b)
        if do_istart and sub == 0:
            _istart(i // CB + 1, 1 - m)
        if do_gstart:
            if do_iwait and (pos + 1) % CB == 0:
                _iwait((i + 1) // CB, ((pos + 1) // CB) % 2)
            _gstart(((pos + 1) // CB) % 2, (pos + 1) % CB, 1 - b)
        _gwait(m, sub, b)
        _scale(m, sub, b)
        _sstart(m, sub, b)

    _istart(jnp.int32(0), 0)
    _iwait(jnp.int32(0), 0)
    _gstart(0, 0, 0)

    _iter(jnp.int32(0), 0, False, True, True, True)

    ngroups = (NCHUNK - 1 - 5) // 6

    def _group(g, _):
        for t in range(6):
            i = 1 + g * 6 + t
            _iter(i, (1 + t) % 6, True, True, True, True)
        return _

    lax.fori_loop(0, ngroups, _group, 0)

    hi = 1 + ngroups * 6
    for i in range(hi, NCHUNK):
        _iter(jnp.int32(i), i % 6,
              True, i // CB + 1 < NB, i + 1 < NCHUNK, True)

    _swait((((NCHUNK - 1) // CB) % 2), (NCHUNK - 1) % CB, (NCHUNK - 1) % 2)

    plsc.subcore_barrier()

    pltpu.sync_copy(acc_sh.at[pl.ds(s * STRIPE, STRIPE)],
                    out_hbm.at[c, pl.ds(s * STRIPE, STRIPE)])

    @pl.when(s == 0)
    def _copy_tail():
        pltpu.sync_copy(acc_sh.at[pl.ds(NS * STRIPE, TAIL)],
                        out_hbm.at[c, pl.ds(NS * STRIPE, TAIL)])


def _tc_body(x_ref, p_ref, bi_col_ref, bi_row_ref, wrel_t_ref, brel_ref,
             wroot_t_ref, gnw_ref, gnb_ref, gnms_ref, out_ref):
    x = x_ref[...]
    agg = p_ref[0] + p_ref[1]
    h = (jnp.dot(agg, wrel_t_ref[...], preferred_element_type=jnp.float32)
         + brel_ref[...]
         + jnp.dot(x, wroot_t_ref[...], preferred_element_type=jnp.float32))
    h = jnp.maximum(h, 0.0)

    bi_col = bi_col_ref[...]
    bi_row = bi_row_ref[...]
    mt = (lax.broadcasted_iota(jnp.int32, (G, N), 0) == bi_row)
    mt = mt.astype(jnp.float32)
    m = (lax.broadcasted_iota(jnp.int32, (N, G), 1) == bi_col)
    m = m.astype(jnp.float32)

    cnt = jnp.maximum(jnp.sum(mt, axis=1, keepdims=True), 1.0)
    mean = jnp.dot(mt, h, preferred_element_type=jnp.float32) / cnt
    ms = mean * gnms_ref[...]
    out = h - jnp.dot(m, ms, preferred_element_type=jnp.float32)
    var = jnp.dot(mt, out * out, preferred_element_type=jnp.float32) / cnt
    rstd = 1.0 / jnp.sqrt(var + EPS)
    out = out * jnp.dot(m, rstd, preferred_element_type=jnp.float32)
    out_ref[...] = out * gnw_ref[...] + gnb_ref[...]


def kernel(x, edge_index, edge_attr, batch_index, W_rel, b_rel, W_root,
           gn_weight, gn_bias, gn_mean_scale):
    pad = EPAD - E
    src = jnp.pad(edge_index[0], (0, pad)).reshape(NW, NB, 1, CB, K)
    dst = jnp.pad(edge_index[1], (0, pad)).reshape(NW, NB, 1, CB, K)
    attr = lax.bitcast_convert_type(jnp.pad(edge_attr, (0, pad)), jnp.int32)
    attr = attr.reshape(NW, NB, 1, CB, K)
    idx = jnp.concatenate([src, dst, attr], axis=2)
    partials = _sc_aggregate(x, idx)

    bi_col = batch_index.reshape(N, 1)
    bi_row = batch_index.reshape(1, N)
    out = pl.pallas_call(
        _tc_body,
        out_shape=jax.ShapeDtypeStruct((N, D), jnp.float32),
    )(x, partials, bi_col, bi_row, W_rel.T, b_rel.reshape(1, D), W_root.T,
      gn_weight.reshape(1, D), gn_bias.reshape(1, D),
      gn_mean_scale.reshape(1, D))
    return out
